# baseline (speedup 1.0000x reference)
import functools

import jax
import jax.numpy as jnp
import numpy as np
from jax import lax
from jax.experimental import pallas as pl
from jax.experimental.pallas import tpu as pltpu
from jax.experimental.pallas import tpu_sc as plsc

D = 256
NH = 8
NL = 4
NP = 4
HD = D // NH
PL = NH * NL * NP
LEVEL_HW = ((32, 32), (16, 16), (8, 8), (4, 4))
HWS = tuple(h * w for h, w in LEVEL_HW)
NTOK = sum(HWS)
BASES = tuple(int(x) for x in np.cumsum((0,) + HWS[:-1]))
B = 2
T = 2
BT = B * T
NW = BT * NH
NPAD = 1408
NCH = 8
QC = NPAD // NCH
QG = QC // 16
ROWW = HD // 2 + 1


def _np_lane_consts():
    lane_l = (np.arange(PL) // NP) % NL
    wl = np.array([LEVEL_HW[l][1] for l in lane_l], np.float32)
    hl = np.array([LEVEL_HW[l][0] for l in lane_l], np.float32)
    base = np.array([BASES[l] for l in lane_l], np.float32)
    return wl[None, :], hl[None, :], base[None, :]


def _np_ref_points():
    xs, ys = [], []
    for (h, w) in LEVEL_HW:
        y, x = np.meshgrid(np.arange(h, dtype=np.float32),
                           np.arange(w, dtype=np.float32), indexing='ij')
        xs.append(((x + 0.5) / w).reshape(-1))
        ys.append(((y + 0.5) / h).reshape(-1))
    return np.concatenate(xs), np.concatenate(ys)


_WL_LANE, _HL_LANE, _BASE_LANE = _np_lane_consts()
_REF_X, _REF_Y = _np_ref_points()
_GX = (_REF_X[:, None] * _WL_LANE - 0.5).astype(np.float32)
_GY = (_REF_Y[:, None] * _HL_LANE - 0.5).astype(np.float32)
_SEG = (np.arange(PL)[:, None] // (NL * NP) ==
        np.arange(PL)[None, :] // (NL * NP)).astype(np.float32)


def _prep_kernel(x_ref, wqt, wvt, woxt, woyt, wat, bq, bv, box, boy, bat,
                 gx, gy, wl, hl, base, seg, v_out, idx_out, w_out):
    x = x_ref[0]
    f32 = jnp.float32
    q = jnp.dot(x, wqt[...], preferred_element_type=f32) + bq[...]
    v_out[0] = jnp.dot(x, wvt[...], preferred_element_type=f32) + bv[...]
    ox = jnp.dot(q, woxt[...], preferred_element_type=f32) + box[...]
    oy = jnp.dot(q, woyt[...], preferred_element_type=f32) + boy[...]
    lg = jnp.dot(q, wat[...], preferred_element_type=f32) + bat[...]
    lg = lg - jnp.max(lg, axis=-1, keepdims=True)
    e = jnp.exp(lg)
    aw = e / jnp.dot(e, seg[...], preferred_element_type=f32)
    xpix = gx[...] + ox
    ypix = gy[...] + oy
    x0 = jnp.floor(xpix)
    y0 = jnp.floor(ypix)
    fx = xpix - x0
    fy = ypix - y0
    wlv = wl[...]
    hlv = hl[...]
    bsv = base[...]
    for ci, (cx, cy) in enumerate(((0, 0), (1, 0), (0, 1), (1, 1))):
        xi = x0 + cx
        yi = y0 + cy
        valid = ((xi >= 0) & (xi <= wlv - 1) & (yi >= 0) & (yi <= hlv - 1))
        xc = jnp.clip(xi, 0.0, wlv - 1)
        yc = jnp.clip(yi, 0.0, hlv - 1)
        idx = (bsv + yc * wlv + xc) * ROWW
        wxc = fx if cx else 1.0 - fx
        wyc = fy if cy else 1.0 - fy
        w = aw * wxc * wyc * valid.astype(f32)
        idx_out[0, ci] = idx.astype(jnp.int32)
        w_out[0, ci] = w


def _run_prep(x_all, WqT, WvT, WoxT, WoyT, WaT, bq, bv, box, boy, bat):
    full = lambda s: pl.BlockSpec(s, lambda i: (0,) * len(s))
    grid = (BT,)
    out_shapes = (
        jax.ShapeDtypeStruct((BT, NTOK, D), jnp.float32),
        jax.ShapeDtypeStruct((BT, 4, NTOK, PL), jnp.int32),
        jax.ShapeDtypeStruct((BT, 4, NTOK, PL), jnp.float32),
    )
    in_specs = [
        pl.BlockSpec((1, NTOK, D), lambda i: (i, 0, 0)),
        full((D, D)), full((D, D)), full((D, PL)), full((D, PL)),
        full((D, PL)),
        full((1, D)), full((1, D)), full((1, PL)), full((1, PL)),
        full((1, PL)),
        full((NTOK, PL)), full((NTOK, PL)),
        full((1, PL)), full((1, PL)), full((1, PL)), full((PL, PL)),
    ]
    out_specs = (
        pl.BlockSpec((1, NTOK, D), lambda i: (i, 0, 0)),
        pl.BlockSpec((1, 4, NTOK, PL), lambda i: (i, 0, 0, 0)),
        pl.BlockSpec((1, 4, NTOK, PL), lambda i: (i, 0, 0, 0)),
    )
    return pl.pallas_call(
        _prep_kernel, grid=grid, in_specs=in_specs, out_specs=out_specs,
        out_shape=out_shapes,
    )(x_all, WqT, WvT, WoxT, WoyT, WaT, bq, bv, box, boy, bat,
      jnp.asarray(_GX), jnp.asarray(_GY),
      jnp.asarray(_WL_LANE), jnp.asarray(_HL_LANE), jnp.asarray(_BASE_LANE),
      jnp.asarray(_SEG))


def _sc_sample_kernel(vh_hbm, idx_hbm, w_hbm, out_hbm,
                      table_v, idx_v, w_v, out_v):
    nc = 2
    wid = lax.axis_index("s") * nc + lax.axis_index("c")
    pltpu.sync_copy(vh_hbm.at[wid], table_v)

    def chunk_body(c, carry):
        pltpu.sync_copy(idx_hbm.at[wid, c], idx_v)
        pltpu.sync_copy(w_hbm.at[wid, c], w_v)

        def group_body(g, carry2):
            qbase = pl.multiple_of(g * 16, 16)

            def dpass(klo):
                def j_body(j, accs):
                    idxv = idx_v[j, pl.ds(qbase, 16)]
                    wv = w_v[j, pl.ds(qbase, 16)]
                    out = []
                    for kk in range(8):
                        word = plsc.load_gather(table_v, [idxv + (klo + kk)])
                        lo = plsc.bitcast(word << 16, jnp.float32)
                        hi = plsc.bitcast(word & jnp.int32(-65536), jnp.float32)
                        out.append(accs[2 * kk] + wv * lo)
                        out.append(accs[2 * kk + 1] + wv * hi)
                    return tuple(out)

                accs = lax.fori_loop(
                    0, 64, j_body,
                    tuple(jnp.zeros((16,), jnp.float32) for _ in range(16)))
                for dd in range(16):
                    out_v[2 * klo + dd, pl.ds(qbase, 16)] = accs[dd]

            dpass(0)
            dpass(8)
            return carry2

        lax.fori_loop(0, QG, group_body, 0)
        pltpu.sync_copy(out_v, out_hbm.at[wid, c])
        return carry

    lax.fori_loop(0, NCH, chunk_body, 0)


def _run_sc_sample(vh, idx, w):
    mesh = plsc.VectorSubcoreMesh(core_axis_name="c", subcore_axis_name="s")
    fn = functools.partial(
        pl.kernel,
        out_type=jax.ShapeDtypeStruct((NW, NCH, HD, QC), jnp.float32),
        mesh=mesh,
        compiler_params=pltpu.CompilerParams(needs_layout_passes=False),
        scratch_types=[
            pltpu.VMEM((NTOK * ROWW,), jnp.int32),
            pltpu.VMEM((64, QC), jnp.int32),
            pltpu.VMEM((64, QC), jnp.float32),
            pltpu.VMEM((HD, QC), jnp.float32),
        ],
    )(_sc_sample_kernel)
    return fn(vh, idx, w)


def _out_proj_kernel(s_ref, wt, b, o_ref):
    o_ref[...] = jnp.dot(s_ref[...], wt[...],
                         preferred_element_type=jnp.float32) + b[...]


def _run_out_proj(s, WoutT, bout):
    return pl.pallas_call(
        _out_proj_kernel,
        grid=(BT,),
        in_specs=[
            pl.BlockSpec((1, NTOK, D), lambda i: (i, 0, 0)),
            pl.BlockSpec((D, D), lambda i: (0, 0)),
            pl.BlockSpec((1, D), lambda i: (0, 0)),
        ],
        out_specs=pl.BlockSpec((1, NTOK, D), lambda i: (i, 0, 0)),
        out_shape=jax.ShapeDtypeStruct((BT, NTOK, D), jnp.float32),
    )(s, WoutT, bout)


def kernel(feat0, feat1, feat2, feat3, Wq, bq, Wv, bv, Woff, boff,
           Wattn, battn, Wout, bout):
    feats = (feat0, feat1, feat2, feat3)
    x_all = jnp.concatenate(
        [f.transpose(0, 4, 2, 3, 1).reshape(B, T, hw, D)
         for f, hw in zip(feats, HWS)], axis=2).reshape(BT, NTOK, D)

    WqT = Wq.T
    WvT = Wv.T
    WoxT = Woff[0::2].T
    WoyT = Woff[1::2].T
    WaT = Wattn.T
    box = boff[0::2][None, :]
    boy = boff[1::2][None, :]

    v, idx, w = _run_prep(
        x_all, WqT, WvT, WoxT, WoyT, WaT,
        bq[None, :], bv[None, :], box, boy, battn[None, :])

    vp = v.reshape(BT, NTOK, NH, HD // 2, 2).astype(jnp.bfloat16)
    vp = jax.lax.bitcast_convert_type(vp, jnp.int32)
    vp = jnp.pad(vp, ((0, 0), (0, 0), (0, 0), (0, 1)))
    vh = vp.transpose(0, 2, 1, 3).reshape(NW, NTOK * ROWW)

    def to_sc(a):
        a = a.reshape(BT, 4, NTOK, NH, NL * NP).transpose(0, 3, 1, 4, 2)
        a = a.reshape(NW, 64, NTOK)
        a = jnp.pad(a, ((0, 0), (0, 0), (0, NPAD - NTOK)))
        return a.reshape(NW, 64, NCH, QC).transpose(0, 2, 1, 3)

    idx_sc = to_sc(idx)
    w_sc = to_sc(w)

    sampled = _run_sc_sample(vh, idx_sc, w_sc)

    s = sampled.reshape(BT, NH, NCH, HD, QC).transpose(0, 2, 4, 1, 3)
    s = s.reshape(BT, NPAD, D)[:, :NTOK]

    y = _run_out_proj(s, Wout.T, bout[None, :])
    y = y.reshape(B, T, NTOK, D)

    outs = []
    for l, (h, wdt) in enumerate(LEVEL_HW):
        seg = y[:, :, BASES[l]:BASES[l] + HWS[l]]
        seg = seg.reshape(B, T, h, wdt, D).transpose(0, 4, 2, 3, 1)
        outs.append(seg)
    return tuple(outs)


# speedup vs baseline: 1.2589x; 1.2589x over previous
import functools

import jax
import jax.numpy as jnp
import numpy as np
from jax import lax
from jax.experimental import pallas as pl
from jax.experimental.pallas import tpu as pltpu
from jax.experimental.pallas import tpu_sc as plsc

D = 256
NH = 8
NL = 4
NP = 4
HD = D // NH
PL = NH * NL * NP
LEVEL_HW = ((32, 32), (16, 16), (8, 8), (4, 4))
HWS = tuple(h * w for h, w in LEVEL_HW)
NTOK = sum(HWS)
BASES = tuple(int(x) for x in np.cumsum((0,) + HWS[:-1]))
B = 2
T = 2
BT = B * T
NW = BT * NH
NPAD = 1408
NCH = 8
QC = NPAD // NCH
QG = QC // 16
ROWW = HD + 1


def _np_lane_consts():
    lane_l = (np.arange(PL) // NP) % NL
    wl = np.array([LEVEL_HW[l][1] for l in lane_l], np.float32)
    hl = np.array([LEVEL_HW[l][0] for l in lane_l], np.float32)
    base = np.array([BASES[l] for l in lane_l], np.float32)
    return wl[None, :], hl[None, :], base[None, :]


def _np_ref_points():
    xs, ys = [], []
    for (h, w) in LEVEL_HW:
        y, x = np.meshgrid(np.arange(h, dtype=np.float32),
                           np.arange(w, dtype=np.float32), indexing='ij')
        xs.append(((x + 0.5) / w).reshape(-1))
        ys.append(((y + 0.5) / h).reshape(-1))
    return np.concatenate(xs), np.concatenate(ys)


_WL_LANE, _HL_LANE, _BASE_LANE = _np_lane_consts()
_REF_X, _REF_Y = _np_ref_points()
_GX = (_REF_X[:, None] * _WL_LANE - 0.5).astype(np.float32)
_GY = (_REF_Y[:, None] * _HL_LANE - 0.5).astype(np.float32)
_SEG = (np.arange(PL)[:, None] // (NL * NP) ==
        np.arange(PL)[None, :] // (NL * NP)).astype(np.float32)


def _prep_kernel(x_ref, wqt, wvt, woxt, woyt, wat, bq, bv, box, boy, bat,
                 gx, gy, wl, hl, base, seg, v_out, iw_out):
    x = x_ref[0]
    f32 = jnp.float32
    q = jnp.dot(x, wqt[...], preferred_element_type=f32) + bq[...]
    v_out[0] = jnp.dot(x, wvt[...], preferred_element_type=f32) + bv[...]
    ox = jnp.dot(q, woxt[...], preferred_element_type=f32) + box[...]
    oy = jnp.dot(q, woyt[...], preferred_element_type=f32) + boy[...]
    lg = jnp.dot(q, wat[...], preferred_element_type=f32) + bat[...]
    lg = lg - jnp.max(lg, axis=-1, keepdims=True)
    e = jnp.exp(lg)
    aw = e / jnp.dot(e, seg[...], preferred_element_type=f32)
    xpix = gx[...] + ox
    ypix = gy[...] + oy
    x0 = jnp.floor(xpix)
    y0 = jnp.floor(ypix)
    fx = xpix - x0
    fy = ypix - y0
    wlv = wl[...]
    hlv = hl[...]
    bsv = base[...]
    for ci, (cx, cy) in enumerate(((0, 0), (1, 0), (0, 1), (1, 1))):
        xi = x0 + cx
        yi = y0 + cy
        valid = ((xi >= 0) & (xi <= wlv - 1) & (yi >= 0) & (yi <= hlv - 1))
        xc = jnp.clip(xi, 0.0, wlv - 1)
        yc = jnp.clip(yi, 0.0, hlv - 1)
        idx = (bsv + yc * wlv + xc) * ROWW
        wxc = fx if cx else 1.0 - fx
        wyc = fy if cy else 1.0 - fy
        w = aw * wxc * wyc * valid.astype(f32)
        wbits = jax.lax.bitcast_convert_type(
            w.astype(jnp.bfloat16), jnp.uint16).astype(jnp.int32)
        iw_out[0, ci] = (wbits << 16) | idx.astype(jnp.int32)


def _run_prep(x_all, WqT, WvT, WoxT, WoyT, WaT, bq, bv, box, boy, bat):
    full = lambda s: pl.BlockSpec(s, lambda i: (0,) * len(s))
    grid = (BT,)
    out_shapes = (
        jax.ShapeDtypeStruct((BT, NTOK, NH * ROWW), jnp.float32),
        jax.ShapeDtypeStruct((BT, 4, NTOK, PL), jnp.int32),
    )
    in_specs = [
        pl.BlockSpec((1, NTOK, D), lambda i: (i, 0, 0)),
        full((D, D)), full((D, NH * ROWW)), full((D, PL)), full((D, PL)),
        full((D, PL)),
        full((1, D)), full((1, NH * ROWW)), full((1, PL)), full((1, PL)),
        full((1, PL)),
        full((NTOK, PL)), full((NTOK, PL)),
        full((1, PL)), full((1, PL)), full((1, PL)), full((PL, PL)),
    ]
    out_specs = (
        pl.BlockSpec((1, NTOK, NH * ROWW), lambda i: (i, 0, 0)),
        pl.BlockSpec((1, 4, NTOK, PL), lambda i: (i, 0, 0, 0)),
    )
    return pl.pallas_call(
        _prep_kernel, grid=grid, in_specs=in_specs, out_specs=out_specs,
        out_shape=out_shapes,
    )(x_all, WqT, WvT, WoxT, WoyT, WaT, bq, bv, box, boy, bat,
      jnp.asarray(_GX), jnp.asarray(_GY),
      jnp.asarray(_WL_LANE), jnp.asarray(_HL_LANE), jnp.asarray(_BASE_LANE),
      jnp.asarray(_SEG))


def _sc_sample_kernel(vh_hbm, iw_hbm, out_hbm, table_v, iw_v, out_v):
    nc = 2
    wid = lax.axis_index("s") * nc + lax.axis_index("c")
    pltpu.sync_copy(vh_hbm.at[wid], table_v)

    def chunk_body(c, carry):
        pltpu.sync_copy(iw_hbm.at[wid, c], iw_v)

        def group_body(g, carry2):
            qbase = pl.multiple_of(g * 16, 16)

            def dpass(dlo):
                def j_body(j, accs):
                    iw = iw_v[j, pl.ds(qbase, 16)]
                    idxv = iw & jnp.int32(0xFFFF)
                    wv = plsc.bitcast(iw & jnp.int32(-65536), jnp.float32)
                    out = []
                    for dd in range(16):
                        col = plsc.load_gather(table_v, [idxv + (dlo + dd)])
                        out.append(accs[dd] + wv * col)
                    return tuple(out)

                accs = lax.fori_loop(
                    0, 64, j_body,
                    tuple(jnp.zeros((16,), jnp.float32) for _ in range(16)))
                for dd in range(16):
                    out_v[dlo + dd, pl.ds(qbase, 16)] = accs[dd]

            dpass(0)
            dpass(16)
            return carry2

        lax.fori_loop(0, QG, group_body, 0)
        pltpu.sync_copy(out_v, out_hbm.at[wid, c])
        return carry

    lax.fori_loop(0, NCH, chunk_body, 0)


def _run_sc_sample(vh, iw):
    mesh = plsc.VectorSubcoreMesh(core_axis_name="c", subcore_axis_name="s")
    fn = functools.partial(
        pl.kernel,
        out_type=jax.ShapeDtypeStruct((NW, NCH, HD, QC), jnp.float32),
        mesh=mesh,
        compiler_params=pltpu.CompilerParams(needs_layout_passes=False),
        scratch_types=[
            pltpu.VMEM((NTOK * ROWW,), jnp.float32),
            pltpu.VMEM((64, QC), jnp.int32),
            pltpu.VMEM((HD, QC), jnp.float32),
        ],
    )(_sc_sample_kernel)
    return fn(vh, iw)


def _out_proj_kernel(s_ref, wt, b, o_ref):
    o_ref[...] = jnp.dot(s_ref[...], wt[...],
                         preferred_element_type=jnp.float32) + b[...]


def _run_out_proj(s, WoutT, bout):
    return pl.pallas_call(
        _out_proj_kernel,
        grid=(BT,),
        in_specs=[
            pl.BlockSpec((1, NTOK, D), lambda i: (i, 0, 0)),
            pl.BlockSpec((D, D), lambda i: (0, 0)),
            pl.BlockSpec((1, D), lambda i: (0, 0)),
        ],
        out_specs=pl.BlockSpec((1, NTOK, D), lambda i: (i, 0, 0)),
        out_shape=jax.ShapeDtypeStruct((BT, NTOK, D), jnp.float32),
    )(s, WoutT, bout)


def kernel(feat0, feat1, feat2, feat3, Wq, bq, Wv, bv, Woff, boff,
           Wattn, battn, Wout, bout):
    feats = (feat0, feat1, feat2, feat3)
    x_all = jnp.concatenate(
        [f.transpose(0, 4, 2, 3, 1).reshape(B, T, hw, D)
         for f, hw in zip(feats, HWS)], axis=2).reshape(BT, NTOK, D)

    WqT = Wq.T
    WvT = jnp.pad(Wv.T.reshape(D, NH, HD), ((0, 0), (0, 0), (0, ROWW - HD)))
    WvT = WvT.reshape(D, NH * ROWW)
    bv_pad = jnp.pad(bv.reshape(NH, HD), ((0, 0), (0, ROWW - HD))).reshape(-1)
    WoxT = Woff[0::2].T
    WoyT = Woff[1::2].T
    WaT = Wattn.T
    box = boff[0::2][None, :]
    boy = boff[1::2][None, :]

    v, iw = _run_prep(
        x_all, WqT, WvT, WoxT, WoyT, WaT,
        bq[None, :], bv_pad[None, :], box, boy, battn[None, :])

    vh = v.reshape(BT, NTOK, NH, ROWW).transpose(0, 2, 1, 3).reshape(
        NW, NTOK * ROWW)

    a = iw.reshape(BT, 4, NTOK, NH, NL * NP).transpose(0, 3, 1, 4, 2)
    a = a.reshape(NW, 64, NTOK)
    a = jnp.pad(a, ((0, 0), (0, 0), (0, NPAD - NTOK)))
    iw_sc = a.reshape(NW, 64, NCH, QC).transpose(0, 2, 1, 3)

    sampled = _run_sc_sample(vh, iw_sc)

    s = sampled.reshape(BT, NH, NCH, HD, QC).transpose(0, 2, 4, 1, 3)
    s = s.reshape(BT, NPAD, D)[:, :NTOK]

    y = _run_out_proj(s, Wout.T, bout[None, :])
    y = y.reshape(B, T, NTOK, D)

    outs = []
    for l, (h, wdt) in enumerate(LEVEL_HW):
        seg = y[:, :, BASES[l]:BASES[l] + HWS[l]]
        seg = seg.reshape(B, T, h, wdt, D).transpose(0, 4, 2, 3, 1)
        outs.append(seg)
    return tuple(outs)


# speedup vs baseline: 1.2850x; 1.0207x over previous
import functools

import jax
import jax.numpy as jnp
import numpy as np
from jax import lax
from jax.experimental import pallas as pl
from jax.experimental.pallas import tpu as pltpu
from jax.experimental.pallas import tpu_sc as plsc

D = 256
NH = 8
NL = 4
NP = 4
HD = D // NH
PL = NH * NL * NP
LEVEL_HW = ((32, 32), (16, 16), (8, 8), (4, 4))
HWS = tuple(h * w for h, w in LEVEL_HW)
NTOK = sum(HWS)
BASES = tuple(int(x) for x in np.cumsum((0,) + HWS[:-1]))
B = 2
T = 2
BT = B * T
NW = BT * NH
NPAD = 1408
NCH = 8
QC = NPAD // NCH
QG = QC // 16
ROWW = HD + 1


def _np_lane_consts():
    lane_l = (np.arange(PL) // NP) % NL
    wl = np.array([LEVEL_HW[l][1] for l in lane_l], np.float32)
    hl = np.array([LEVEL_HW[l][0] for l in lane_l], np.float32)
    base = np.array([BASES[l] for l in lane_l], np.float32)
    return wl[None, :], hl[None, :], base[None, :]


def _np_ref_points():
    xs, ys = [], []
    for (h, w) in LEVEL_HW:
        y, x = np.meshgrid(np.arange(h, dtype=np.float32),
                           np.arange(w, dtype=np.float32), indexing='ij')
        xs.append(((x + 0.5) / w).reshape(-1))
        ys.append(((y + 0.5) / h).reshape(-1))
    return np.concatenate(xs), np.concatenate(ys)


_WL_LANE, _HL_LANE, _BASE_LANE = _np_lane_consts()
_REF_X, _REF_Y = _np_ref_points()
_GX = (_REF_X[:, None] * _WL_LANE - 0.5).astype(np.float32)
_GY = (_REF_Y[:, None] * _HL_LANE - 0.5).astype(np.float32)
_SEG = (np.arange(PL)[:, None] // (NL * NP) ==
        np.arange(PL)[None, :] // (NL * NP)).astype(np.float32)


def _prep_kernel(x_ref, wq, wvt, wox, woy, wa, bq, bv, box, boy, bat,
                 gx, gy, wl, hl, base, seg, v_out, iw_out):
    x = x_ref[0]
    f32 = jnp.float32
    q = jnp.dot(wq[...], x, preferred_element_type=f32) + bq[...]
    v_out[0] = jax.lax.dot_general(
        x, wvt[...], (((0,), (0,)), ((), ())),
        preferred_element_type=f32) + bv[...]
    ox = jnp.dot(wox[...], q, preferred_element_type=f32) + box[...]
    oy = jnp.dot(woy[...], q, preferred_element_type=f32) + boy[...]
    lg = jnp.dot(wa[...], q, preferred_element_type=f32) + bat[...]
    lg = lg - jnp.max(lg, axis=0, keepdims=True)
    e = jnp.exp(lg)
    aw = e / jnp.dot(seg[...], e, preferred_element_type=f32)
    xpix = gx[...] + ox
    ypix = gy[...] + oy
    x0 = jnp.floor(xpix)
    y0 = jnp.floor(ypix)
    fx = xpix - x0
    fy = ypix - y0
    wlv = wl[...]
    hlv = hl[...]
    bsv = base[...]
    for ci, (cx, cy) in enumerate(((0, 0), (1, 0), (0, 1), (1, 1))):
        xi = x0 + cx
        yi = y0 + cy
        valid = ((xi >= 0) & (xi <= wlv - 1) & (yi >= 0) & (yi <= hlv - 1))
        xc = jnp.clip(xi, 0.0, wlv - 1)
        yc = jnp.clip(yi, 0.0, hlv - 1)
        idx = (bsv + yc * wlv + xc) * ROWW
        wxc = fx if cx else 1.0 - fx
        wyc = fy if cy else 1.0 - fy
        w = aw * wxc * wyc * valid.astype(f32)
        wbits = jax.lax.bitcast_convert_type(
            w.astype(jnp.bfloat16), jnp.uint16).astype(jnp.int32)
        iw_out[0, ci] = (wbits << 16) | idx.astype(jnp.int32)


def _run_prep(x_all, WqT, WvT, WoxT, WoyT, WaT, bq, bv, box, boy, bat):
    full = lambda s: pl.BlockSpec(s, lambda i: (0,) * len(s))
    grid = (BT,)
    out_shapes = (
        jax.ShapeDtypeStruct((BT, NTOK, NH * ROWW), jnp.float32),
        jax.ShapeDtypeStruct((BT, 4, PL, NTOK), jnp.int32),
    )
    in_specs = [
        pl.BlockSpec((1, D, NTOK), lambda i: (i, 0, 0)),
        full((D, D)), full((D, NH * ROWW)), full((PL, D)), full((PL, D)),
        full((PL, D)),
        full((D, 1)), full((1, NH * ROWW)), full((PL, 1)), full((PL, 1)),
        full((PL, 1)),
        full((PL, NTOK)), full((PL, NTOK)),
        full((PL, 1)), full((PL, 1)), full((PL, 1)), full((PL, PL)),
    ]
    out_specs = (
        pl.BlockSpec((1, NTOK, NH * ROWW), lambda i: (i, 0, 0)),
        pl.BlockSpec((1, 4, PL, NTOK), lambda i: (i, 0, 0, 0)),
    )
    return pl.pallas_call(
        _prep_kernel, grid=grid, in_specs=in_specs, out_specs=out_specs,
        out_shape=out_shapes,
    )(x_all, WqT, WvT, WoxT, WoyT, WaT, bq, bv, box, boy, bat,
      jnp.asarray(_GX.T.copy()), jnp.asarray(_GY.T.copy()),
      jnp.asarray(_WL_LANE.T.copy()), jnp.asarray(_HL_LANE.T.copy()),
      jnp.asarray(_BASE_LANE.T.copy()), jnp.asarray(_SEG))


def _sc_sample_kernel(vh_hbm, iw_hbm, out_hbm, table_v, iw_v, out_v):
    nc = 2
    wid = lax.axis_index("s") * nc + lax.axis_index("c")
    pltpu.sync_copy(vh_hbm.at[wid], table_v)

    def chunk_body(c, carry):
        pltpu.sync_copy(iw_hbm.at[wid, c], iw_v)

        def group_body(g, carry2):
            qbase = pl.multiple_of(g * 16, 16)

            def dpass(dlo):
                def j_body(j, accs):
                    iw = iw_v[j, pl.ds(qbase, 16)]
                    idxv = iw & jnp.int32(0xFFFF)
                    wv = plsc.bitcast(iw & jnp.int32(-65536), jnp.float32)
                    out = []
                    for dd in range(16):
                        col = plsc.load_gather(table_v, [idxv + (dlo + dd)])
                        out.append(accs[dd] + wv * col)
                    return tuple(out)

                accs = lax.fori_loop(
                    0, 64, j_body,
                    tuple(jnp.zeros((16,), jnp.float32) for _ in range(16)))
                for dd in range(16):
                    out_v[dlo + dd, pl.ds(qbase, 16)] = accs[dd]

            dpass(0)
            dpass(16)
            return carry2

        lax.fori_loop(0, QG, group_body, 0)
        pltpu.sync_copy(out_v, out_hbm.at[wid, c])
        return carry

    lax.fori_loop(0, NCH, chunk_body, 0)


def _run_sc_sample(vh, iw):
    mesh = plsc.VectorSubcoreMesh(core_axis_name="c", subcore_axis_name="s")
    fn = functools.partial(
        pl.kernel,
        out_type=jax.ShapeDtypeStruct((NW, NCH, HD, QC), jnp.float32),
        mesh=mesh,
        compiler_params=pltpu.CompilerParams(needs_layout_passes=False),
        scratch_types=[
            pltpu.VMEM((NTOK * ROWW,), jnp.float32),
            pltpu.VMEM((64, QC), jnp.int32),
            pltpu.VMEM((HD, QC), jnp.float32),
        ],
    )(_sc_sample_kernel)
    return fn(vh, iw)


def _out_proj_kernel(s_ref, wt, b, o_ref):
    o_ref[0] = jnp.dot(wt[...], s_ref[0],
                       preferred_element_type=jnp.float32) + b[...]


def _run_out_proj(s, Wout, bout_col):
    return pl.pallas_call(
        _out_proj_kernel,
        grid=(BT,),
        in_specs=[
            pl.BlockSpec((1, D, NTOK), lambda i: (i, 0, 0)),
            pl.BlockSpec((D, D), lambda i: (0, 0)),
            pl.BlockSpec((D, 1), lambda i: (0, 0)),
        ],
        out_specs=pl.BlockSpec((1, D, NTOK), lambda i: (i, 0, 0)),
        out_shape=jax.ShapeDtypeStruct((BT, D, NTOK), jnp.float32),
    )(s, Wout, bout_col)


def kernel(feat0, feat1, feat2, feat3, Wq, bq, Wv, bv, Woff, boff,
           Wattn, battn, Wout, bout):
    feats = (feat0, feat1, feat2, feat3)
    x_all = jnp.concatenate(
        [f.transpose(0, 4, 1, 2, 3).reshape(B, T, D, hw)
         for f, hw in zip(feats, HWS)], axis=3).reshape(BT, D, NTOK)

    WvT = jnp.pad(Wv.T.reshape(D, NH, HD), ((0, 0), (0, 0), (0, ROWW - HD)))
    WvT = WvT.reshape(D, NH * ROWW)
    bv_pad = jnp.pad(bv.reshape(NH, HD), ((0, 0), (0, ROWW - HD))).reshape(-1)
    Wox = Woff[0::2]
    Woy = Woff[1::2]

    v, iw = _run_prep(
        x_all, Wq, WvT, Wox, Woy, Wattn,
        bq[:, None], bv_pad[None, :], boff[0::2][:, None],
        boff[1::2][:, None], battn[:, None])

    vh = v.reshape(BT, NTOK, NH, ROWW).transpose(0, 2, 1, 3).reshape(
        NW, NTOK * ROWW)

    a = iw.reshape(BT, 4, NH, NL * NP, NTOK).transpose(0, 2, 1, 3, 4)
    a = a.reshape(NW, 64, NTOK)
    a = jnp.pad(a, ((0, 0), (0, 0), (0, NPAD - NTOK)))
    iw_sc = a.reshape(NW, 64, NCH, QC).transpose(0, 2, 1, 3)

    sampled = _run_sc_sample(vh, iw_sc)

    s = sampled.reshape(BT, NH, NCH, HD, QC).transpose(0, 1, 3, 2, 4)
    s = s.reshape(BT, D, NPAD)[:, :, :NTOK]

    y = _run_out_proj(s, Wout, bout[:, None])
    y = y.reshape(B, T, D, NTOK)

    outs = []
    for l, (h, wdt) in enumerate(LEVEL_HW):
        seg = y[:, :, :, BASES[l]:BASES[l] + HWS[l]]
        seg = seg.reshape(B, T, D, h, wdt).transpose(0, 2, 3, 4, 1)
        outs.append(seg)
    return tuple(outs)


# speedup vs baseline: 1.5519x; 1.2077x over previous
import functools

import jax
import jax.numpy as jnp
import numpy as np
from jax import lax
from jax.experimental import pallas as pl
from jax.experimental.pallas import tpu as pltpu
from jax.experimental.pallas import tpu_sc as plsc

D = 256
NH = 8
NL = 4
NP = 4
HD = D // NH
PL = NH * NL * NP
LEVEL_HW = ((32, 32), (16, 16), (8, 8), (4, 4))
HWS = tuple(h * w for h, w in LEVEL_HW)
NTOK = sum(HWS)
BASES = tuple(int(x) for x in np.cumsum((0,) + HWS[:-1]))
B = 2
T = 2
BT = B * T
NW = BT * NH
NPAD = 1408
NCH = 11
QC = NPAD // NCH
QG = QC // 16
ROWW = HD + 1


def _np_lane_consts():
    lane_l = (np.arange(PL) // NP) % NL
    wl = np.array([LEVEL_HW[l][1] for l in lane_l], np.float32)
    hl = np.array([LEVEL_HW[l][0] for l in lane_l], np.float32)
    base = np.array([BASES[l] for l in lane_l], np.float32)
    return wl[None, :], hl[None, :], base[None, :]


def _np_ref_points():
    xs, ys = [], []
    for (h, w) in LEVEL_HW:
        y, x = np.meshgrid(np.arange(h, dtype=np.float32),
                           np.arange(w, dtype=np.float32), indexing='ij')
        xs.append(((x + 0.5) / w).reshape(-1))
        ys.append(((y + 0.5) / h).reshape(-1))
    return np.concatenate(xs), np.concatenate(ys)


_WL_LANE, _HL_LANE, _BASE_LANE = _np_lane_consts()
_REF_X, _REF_Y = _np_ref_points()
_GX = (_REF_X[:, None] * _WL_LANE - 0.5).astype(np.float32)
_GY = (_REF_Y[:, None] * _HL_LANE - 0.5).astype(np.float32)
_SEG = (np.arange(PL)[:, None] // (NL * NP) ==
        np.arange(PL)[None, :] // (NL * NP)).astype(np.float32)


def _prep_kernel(x_ref, wq, wvt, wox, woy, wa, bq, bv, box, boy, bat,
                 gx, gy, wl, hl, base, seg, v_out, iw_out):
    x = x_ref[0]
    f32 = jnp.float32
    q = jnp.dot(wq[...], x, preferred_element_type=f32) + bq[...]
    vfull = jax.lax.dot_general(
        x, wvt[...], (((0,), (0,)), ((), ())),
        preferred_element_type=f32) + bv[...]
    for h in range(NH):
        v_out[0, h] = vfull[:, h * ROWW:(h + 1) * ROWW]
    ox = jnp.dot(wox[...], q, preferred_element_type=f32) + box[...]
    oy = jnp.dot(woy[...], q, preferred_element_type=f32) + boy[...]
    lg = jnp.dot(wa[...], q, preferred_element_type=f32) + bat[...]
    lg = lg - jnp.max(lg, axis=0, keepdims=True)
    e = jnp.exp(lg)
    aw = e / jnp.dot(seg[...], e, preferred_element_type=f32)
    xpix = gx[...] + ox
    ypix = gy[...] + oy
    x0 = jnp.floor(xpix)
    y0 = jnp.floor(ypix)
    fx = xpix - x0
    fy = ypix - y0
    wlv = wl[...]
    hlv = hl[...]
    bsv = base[...]
    for ci, (cx, cy) in enumerate(((0, 0), (1, 0), (0, 1), (1, 1))):
        xi = x0 + cx
        yi = y0 + cy
        valid = ((xi >= 0) & (xi <= wlv - 1) & (yi >= 0) & (yi <= hlv - 1))
        xc = jnp.clip(xi, 0.0, wlv - 1)
        yc = jnp.clip(yi, 0.0, hlv - 1)
        idx = (bsv + yc * wlv + xc) * ROWW
        wxc = fx if cx else 1.0 - fx
        wyc = fy if cy else 1.0 - fy
        w = aw * wxc * wyc * valid.astype(f32)
        wbits = jax.lax.bitcast_convert_type(
            w.astype(jnp.bfloat16), jnp.uint16).astype(jnp.int32)
        iw_out[0, ci] = (wbits << 16) | idx.astype(jnp.int32)


def _run_prep(x_all, WqT, WvT, WoxT, WoyT, WaT, bq, bv, box, boy, bat):
    full = lambda s: pl.BlockSpec(s, lambda i: (0,) * len(s))
    grid = (BT,)
    out_shapes = (
        jax.ShapeDtypeStruct((BT, NH, NTOK, ROWW), jnp.float32),
        jax.ShapeDtypeStruct((BT, 4, PL, NTOK), jnp.int32),
    )
    in_specs = [
        pl.BlockSpec((1, D, NTOK), lambda i: (i, 0, 0)),
        full((D, D)), full((D, NH * ROWW)), full((PL, D)), full((PL, D)),
        full((PL, D)),
        full((D, 1)), full((1, NH * ROWW)), full((PL, 1)), full((PL, 1)),
        full((PL, 1)),
        full((PL, NTOK)), full((PL, NTOK)),
        full((PL, 1)), full((PL, 1)), full((PL, 1)), full((PL, PL)),
    ]
    out_specs = (
        pl.BlockSpec((1, NH, NTOK, ROWW), lambda i: (i, 0, 0, 0)),
        pl.BlockSpec((1, 4, PL, NTOK), lambda i: (i, 0, 0, 0)),
    )
    return pl.pallas_call(
        _prep_kernel, grid=grid, in_specs=in_specs, out_specs=out_specs,
        out_shape=out_shapes,
    )(x_all, WqT, WvT, WoxT, WoyT, WaT, bq, bv, box, boy, bat,
      jnp.asarray(_GX.T.copy()), jnp.asarray(_GY.T.copy()),
      jnp.asarray(_WL_LANE.T.copy()), jnp.asarray(_HL_LANE.T.copy()),
      jnp.asarray(_BASE_LANE.T.copy()), jnp.asarray(_SEG))


def _sc_sample_kernel(vh_hbm, iw_hbm, out_hbm, table_v, iw_v, out_v):
    nc = 2
    wid = lax.axis_index("s") * nc + lax.axis_index("c")
    pltpu.sync_copy(vh_hbm.at[wid], table_v)

    def chunk_body(c, carry):
        qoff = pl.multiple_of(c * QC, QC)
        pltpu.sync_copy(iw_hbm.at[wid, :, pl.ds(qoff, QC)], iw_v)

        def group_body(g, carry2):
            qbase = pl.multiple_of(g * 16, 16)

            def dpass(dlo):
                def j_body(j, accs):
                    iw = iw_v[j, pl.ds(qbase, 16)]
                    idxv = iw & jnp.int32(0xFFFF)
                    wv = plsc.bitcast(iw & jnp.int32(-65536), jnp.float32)
                    out = []
                    for dd in range(16):
                        col = plsc.load_gather(table_v, [idxv + (dlo + dd)])
                        out.append(accs[dd] + wv * col)
                    return tuple(out)

                accs = lax.fori_loop(
                    0, 64, j_body,
                    tuple(jnp.zeros((16,), jnp.float32) for _ in range(16)))
                for dd in range(16):
                    out_v[dlo + dd, pl.ds(qbase, 16)] = accs[dd]

            dpass(0)
            dpass(16)
            return carry2

        lax.fori_loop(0, QG, group_body, 0)
        pltpu.sync_copy(out_v, out_hbm.at[wid, :, pl.ds(qoff, QC)])
        return carry

    lax.fori_loop(0, NCH, chunk_body, 0)


def _run_sc_sample(vh, iw):
    mesh = plsc.VectorSubcoreMesh(core_axis_name="c", subcore_axis_name="s")
    fn = functools.partial(
        pl.kernel,
        out_type=jax.ShapeDtypeStruct((NW, HD, NPAD), jnp.float32),
        mesh=mesh,
        compiler_params=pltpu.CompilerParams(needs_layout_passes=False),
        scratch_types=[
            pltpu.VMEM((NTOK * ROWW,), jnp.float32),
            pltpu.VMEM((64, QC), jnp.int32),
            pltpu.VMEM((HD, QC), jnp.float32),
        ],
    )(_sc_sample_kernel)
    return fn(vh, iw)


def _out_proj_kernel(s_ref, wt, b, o_ref):
    o_ref[0] = jnp.dot(wt[...], s_ref[0],
                       preferred_element_type=jnp.float32) + b[...]


def _run_out_proj(s, Wout, bout_col):
    return pl.pallas_call(
        _out_proj_kernel,
        grid=(BT,),
        in_specs=[
            pl.BlockSpec((1, D, NTOK), lambda i: (i, 0, 0)),
            pl.BlockSpec((D, D), lambda i: (0, 0)),
            pl.BlockSpec((D, 1), lambda i: (0, 0)),
        ],
        out_specs=pl.BlockSpec((1, D, NTOK), lambda i: (i, 0, 0)),
        out_shape=jax.ShapeDtypeStruct((BT, D, NTOK), jnp.float32),
    )(s, Wout, bout_col)


def kernel(feat0, feat1, feat2, feat3, Wq, bq, Wv, bv, Woff, boff,
           Wattn, battn, Wout, bout):
    feats = (feat0, feat1, feat2, feat3)
    x_all = jnp.concatenate(
        [f.transpose(0, 4, 1, 2, 3).reshape(B, T, D, hw)
         for f, hw in zip(feats, HWS)], axis=3).reshape(BT, D, NTOK)

    WvT = jnp.pad(Wv.T.reshape(D, NH, HD), ((0, 0), (0, 0), (0, ROWW - HD)))
    WvT = WvT.reshape(D, NH * ROWW)
    bv_pad = jnp.pad(bv.reshape(NH, HD), ((0, 0), (0, ROWW - HD))).reshape(-1)
    Wox = Woff[0::2]
    Woy = Woff[1::2]

    v, iw = _run_prep(
        x_all, Wq, WvT, Wox, Woy, Wattn,
        bq[:, None], bv_pad[None, :], boff[0::2][:, None],
        boff[1::2][:, None], battn[:, None])

    vh = v.reshape(NW, NTOK * ROWW)

    a = iw.reshape(BT, 4, NH, NL * NP, NTOK).transpose(0, 2, 1, 3, 4)
    a = a.reshape(NW, 64, NTOK)
    iw_sc = jnp.pad(a, ((0, 0), (0, 0), (0, NPAD - NTOK)))

    sampled = _run_sc_sample(vh, iw_sc)

    s = sampled.reshape(BT, D, NPAD)[:, :, :NTOK]

    y = _run_out_proj(s, Wout, bout[:, None])
    y = y.reshape(B, T, D, NTOK)

    outs = []
    for l, (h, wdt) in enumerate(LEVEL_HW):
        seg = y[:, :, :, BASES[l]:BASES[l] + HWS[l]]
        seg = seg.reshape(B, T, D, h, wdt).transpose(0, 2, 3, 4, 1)
        outs.append(seg)
    return tuple(outs)


# speedup vs baseline: 1.6451x; 1.0601x over previous
import functools

import jax
import jax.numpy as jnp
import numpy as np
from jax import lax
from jax.experimental import pallas as pl
from jax.experimental.pallas import tpu as pltpu
from jax.experimental.pallas import tpu_sc as plsc

D = 256
NH = 8
NL = 4
NP = 4
HD = D // NH
PL = NH * NL * NP
LEVEL_HW = ((32, 32), (16, 16), (8, 8), (4, 4))
HWS = tuple(h * w for h, w in LEVEL_HW)
NTOK = sum(HWS)
BASES = tuple(int(x) for x in np.cumsum((0,) + HWS[:-1]))
B = 2
T = 2
BT = B * T
NW = BT * NH
NPAD = 1408
NCH = 11
QC = NPAD // NCH
QG = QC // 16
ROWW = HD + 1


def _np_lane_consts():
    lane_l = (np.arange(PL) // NP) % NL
    wl = np.array([LEVEL_HW[l][1] for l in lane_l], np.float32)
    hl = np.array([LEVEL_HW[l][0] for l in lane_l], np.float32)
    base = np.array([BASES[l] for l in lane_l], np.float32)
    return wl[None, :], hl[None, :], base[None, :]


def _np_ref_points():
    xs, ys = [], []
    for (h, w) in LEVEL_HW:
        y, x = np.meshgrid(np.arange(h, dtype=np.float32),
                           np.arange(w, dtype=np.float32), indexing='ij')
        xs.append(((x + 0.5) / w).reshape(-1))
        ys.append(((y + 0.5) / h).reshape(-1))
    return np.concatenate(xs), np.concatenate(ys)


_WL_LANE, _HL_LANE, _BASE_LANE = _np_lane_consts()
_REF_X, _REF_Y = _np_ref_points()
_GX = (_REF_X[:, None] * _WL_LANE - 0.5).astype(np.float32)
_GY = (_REF_Y[:, None] * _HL_LANE - 0.5).astype(np.float32)
_SEG = (np.arange(PL)[:, None] // (NL * NP) ==
        np.arange(PL)[None, :] // (NL * NP)).astype(np.float32)


def _prep_kernel(x_ref, wq, wvt, wox, woy, wa, bq, bv, box, boy, bat,
                 gx, gy, wl, hl, base, seg, v_out, iw_out):
    x = x_ref[0]
    f32 = jnp.float32
    q = jnp.dot(wq[...], x, preferred_element_type=f32) + bq[...]
    vfull = jax.lax.dot_general(
        x, wvt[...], (((0,), (0,)), ((), ())),
        preferred_element_type=f32) + bv[...]
    for h in range(NH):
        v_out[0, h] = vfull[:, h * ROWW:(h + 1) * ROWW]
    ox = jnp.dot(wox[...], q, preferred_element_type=f32) + box[...]
    oy = jnp.dot(woy[...], q, preferred_element_type=f32) + boy[...]
    lg = jnp.dot(wa[...], q, preferred_element_type=f32) + bat[...]
    lg = lg - jnp.max(lg, axis=0, keepdims=True)
    e = jnp.exp(lg)
    aw = e / jnp.dot(seg[...], e, preferred_element_type=f32)
    xpix = gx[...] + ox
    ypix = gy[...] + oy
    x0 = jnp.floor(xpix)
    y0 = jnp.floor(ypix)
    fx = xpix - x0
    fy = ypix - y0
    wlv = wl[...]
    hlv = hl[...]
    bsv = base[...]
    for ci, (cx, cy) in enumerate(((0, 0), (1, 0), (0, 1), (1, 1))):
        xi = x0 + cx
        yi = y0 + cy
        valid = ((xi >= 0) & (xi <= wlv - 1) & (yi >= 0) & (yi <= hlv - 1))
        xc = jnp.clip(xi, 0.0, wlv - 1)
        yc = jnp.clip(yi, 0.0, hlv - 1)
        idx = (bsv + yc * wlv + xc) * ROWW
        wxc = fx if cx else 1.0 - fx
        wyc = fy if cy else 1.0 - fy
        w = aw * wxc * wyc * valid.astype(f32)
        wbits = jax.lax.bitcast_convert_type(
            w.astype(jnp.bfloat16), jnp.uint16).astype(jnp.int32)
        comb = (wbits << 16) | idx.astype(jnp.int32)
        comb = jnp.pad(comb, ((0, 0), (0, NPAD - NTOK)))
        for h in range(NH):
            iw_out[0, h, ci] = comb[h * 16:(h + 1) * 16]


def _run_prep(x_all, WqT, WvT, WoxT, WoyT, WaT, bq, bv, box, boy, bat):
    full = lambda s: pl.BlockSpec(s, lambda i: (0,) * len(s))
    grid = (BT,)
    out_shapes = (
        jax.ShapeDtypeStruct((BT, NH, NTOK, ROWW), jnp.float32),
        jax.ShapeDtypeStruct((BT, NH, 4, 16, NPAD), jnp.int32),
    )
    in_specs = [
        pl.BlockSpec((1, D, NTOK), lambda i: (i, 0, 0)),
        full((D, D)), full((D, NH * ROWW)), full((PL, D)), full((PL, D)),
        full((PL, D)),
        full((D, 1)), full((1, NH * ROWW)), full((PL, 1)), full((PL, 1)),
        full((PL, 1)),
        full((PL, NTOK)), full((PL, NTOK)),
        full((PL, 1)), full((PL, 1)), full((PL, 1)), full((PL, PL)),
    ]
    out_specs = (
        pl.BlockSpec((1, NH, NTOK, ROWW), lambda i: (i, 0, 0, 0)),
        pl.BlockSpec((1, NH, 4, 16, NPAD), lambda i: (i, 0, 0, 0, 0)),
    )
    return pl.pallas_call(
        _prep_kernel, grid=grid, in_specs=in_specs, out_specs=out_specs,
        out_shape=out_shapes,
    )(x_all, WqT, WvT, WoxT, WoyT, WaT, bq, bv, box, boy, bat,
      jnp.asarray(_GX.T.copy()), jnp.asarray(_GY.T.copy()),
      jnp.asarray(_WL_LANE.T.copy()), jnp.asarray(_HL_LANE.T.copy()),
      jnp.asarray(_BASE_LANE.T.copy()), jnp.asarray(_SEG))


def _sc_sample_kernel(vh_hbm, iw_hbm, out_hbm, table_v, iw_v, out_v):
    nc = 2
    wid = lax.axis_index("s") * nc + lax.axis_index("c")
    pltpu.sync_copy(vh_hbm.at[wid], table_v)

    def chunk_body(c, carry):
        qoff = pl.multiple_of(c * QC, QC)
        pltpu.sync_copy(iw_hbm.at[wid, :, pl.ds(qoff, QC)], iw_v)

        def group_body(g, carry2):
            qbase = pl.multiple_of(g * 16, 16)

            def dpass(dlo):
                def j_body(j, accs):
                    iw = iw_v[j, pl.ds(qbase, 16)]
                    idxv = iw & jnp.int32(0xFFFF)
                    wv = plsc.bitcast(iw & jnp.int32(-65536), jnp.float32)
                    out = []
                    for dd in range(16):
                        col = plsc.load_gather(table_v, [idxv + (dlo + dd)])
                        out.append(accs[dd] + wv * col)
                    return tuple(out)

                accs = lax.fori_loop(
                    0, 64, j_body,
                    tuple(jnp.zeros((16,), jnp.float32) for _ in range(16)))
                for dd in range(16):
                    out_v[dlo + dd, pl.ds(qbase, 16)] = accs[dd]

            dpass(0)
            dpass(16)
            return carry2

        lax.fori_loop(0, QG, group_body, 0)
        pltpu.sync_copy(out_v, out_hbm.at[wid, :, pl.ds(qoff, QC)])
        return carry

    lax.fori_loop(0, NCH, chunk_body, 0)


def _run_sc_sample(vh, iw):
    mesh = plsc.VectorSubcoreMesh(core_axis_name="c", subcore_axis_name="s")
    fn = functools.partial(
        pl.kernel,
        out_type=jax.ShapeDtypeStruct((NW, HD, NPAD), jnp.float32),
        mesh=mesh,
        compiler_params=pltpu.CompilerParams(needs_layout_passes=False),
        scratch_types=[
            pltpu.VMEM((NTOK * ROWW,), jnp.float32),
            pltpu.VMEM((64, QC), jnp.int32),
            pltpu.VMEM((HD, QC), jnp.float32),
        ],
    )(_sc_sample_kernel)
    return fn(vh, iw)


def _out_proj_kernel(s_ref, wt, b, o_ref):
    o_ref[0] = jnp.dot(wt[...], s_ref[0],
                       preferred_element_type=jnp.float32) + b[...]


def _run_out_proj(s, Wout, bout_col):
    return pl.pallas_call(
        _out_proj_kernel,
        grid=(BT,),
        in_specs=[
            pl.BlockSpec((1, D, NTOK), lambda i: (i, 0, 0)),
            pl.BlockSpec((D, D), lambda i: (0, 0)),
            pl.BlockSpec((D, 1), lambda i: (0, 0)),
        ],
        out_specs=pl.BlockSpec((1, D, NTOK), lambda i: (i, 0, 0)),
        out_shape=jax.ShapeDtypeStruct((BT, D, NTOK), jnp.float32),
    )(s, Wout, bout_col)


def kernel(feat0, feat1, feat2, feat3, Wq, bq, Wv, bv, Woff, boff,
           Wattn, battn, Wout, bout):
    feats = (feat0, feat1, feat2, feat3)
    x_all = jnp.concatenate(
        [f.transpose(0, 4, 1, 2, 3).reshape(B, T, D, hw)
         for f, hw in zip(feats, HWS)], axis=3).reshape(BT, D, NTOK)

    WvT = jnp.pad(Wv.T.reshape(D, NH, HD), ((0, 0), (0, 0), (0, ROWW - HD)))
    WvT = WvT.reshape(D, NH * ROWW)
    bv_pad = jnp.pad(bv.reshape(NH, HD), ((0, 0), (0, ROWW - HD))).reshape(-1)
    Wox = Woff[0::2]
    Woy = Woff[1::2]

    v, iw = _run_prep(
        x_all, Wq, WvT, Wox, Woy, Wattn,
        bq[:, None], bv_pad[None, :], boff[0::2][:, None],
        boff[1::2][:, None], battn[:, None])

    vh = v.reshape(NW, NTOK * ROWW)

    iw_sc = iw.reshape(NW, 64, NPAD)

    sampled = _run_sc_sample(vh, iw_sc)

    s = sampled.reshape(BT, D, NPAD)[:, :, :NTOK]

    y = _run_out_proj(s, Wout, bout[:, None])
    y = y.reshape(B, T, D, NTOK)

    outs = []
    for l, (h, wdt) in enumerate(LEVEL_HW):
        seg = y[:, :, :, BASES[l]:BASES[l] + HWS[l]]
        seg = seg.reshape(B, T, D, h, wdt).transpose(0, 2, 3, 4, 1)
        outs.append(seg)
    return tuple(outs)


# speedup vs baseline: 1.7056x; 1.0368x over previous
import functools

import jax
import jax.numpy as jnp
import numpy as np
from jax import lax
from jax.experimental import pallas as pl
from jax.experimental.pallas import tpu as pltpu
from jax.experimental.pallas import tpu_sc as plsc

D = 256
NH = 8
NL = 4
NP = 4
HD = D // NH
PL = NH * NL * NP
LEVEL_HW = ((32, 32), (16, 16), (8, 8), (4, 4))
HWS = tuple(h * w for h, w in LEVEL_HW)
NTOK = sum(HWS)
BASES = tuple(int(x) for x in np.cumsum((0,) + HWS[:-1]))
B = 2
T = 2
BT = B * T
NW = BT * NH
NPAD = 1408
NCH = 11
QC = NPAD // NCH
QG = QC // 16
ROWW = HD + 1


def _np_lane_consts():
    lane_l = (np.arange(PL) // NP) % NL
    wl = np.array([LEVEL_HW[l][1] for l in lane_l], np.float32)
    hl = np.array([LEVEL_HW[l][0] for l in lane_l], np.float32)
    base = np.array([BASES[l] for l in lane_l], np.float32)
    return wl[None, :], hl[None, :], base[None, :]


def _np_ref_points():
    xs, ys = [], []
    for (h, w) in LEVEL_HW:
        y, x = np.meshgrid(np.arange(h, dtype=np.float32),
                           np.arange(w, dtype=np.float32), indexing='ij')
        xs.append(((x + 0.5) / w).reshape(-1))
        ys.append(((y + 0.5) / h).reshape(-1))
    return np.concatenate(xs), np.concatenate(ys)


_WL_LANE, _HL_LANE, _BASE_LANE = _np_lane_consts()
_REF_X, _REF_Y = _np_ref_points()
_GX = (_REF_X[:, None] * _WL_LANE - 0.5).astype(np.float32)
_GY = (_REF_Y[:, None] * _HL_LANE - 0.5).astype(np.float32)
_SEG = (np.arange(PL)[:, None] // (NL * NP) ==
        np.arange(PL)[None, :] // (NL * NP)).astype(np.float32)


def _prep_kernel(x_ref, wq, wvt, wox, woy, wa, bq, bv, box, boy, bat,
                 gx, gy, wl, hl, base, seg, v_out, iw_out):
    x = x_ref[0]
    f32 = jnp.float32
    q = jnp.dot(wq[...], x, preferred_element_type=f32) + bq[...]
    vfull = jax.lax.dot_general(
        x, wvt[...], (((0,), (0,)), ((), ())),
        preferred_element_type=f32) + bv[...]
    for h in range(NH):
        v_out[0, h] = vfull[:, h * ROWW:(h + 1) * ROWW]
    ox = jnp.dot(wox[...], q, preferred_element_type=f32) + box[...]
    oy = jnp.dot(woy[...], q, preferred_element_type=f32) + boy[...]
    lg = jnp.dot(wa[...], q, preferred_element_type=f32) + bat[...]
    lg = lg - jnp.max(lg, axis=0, keepdims=True)
    e = jnp.exp(lg)
    aw = e / jnp.dot(seg[...], e, preferred_element_type=f32)
    xpix = gx[...] + ox
    ypix = gy[...] + oy
    x0 = jnp.floor(xpix)
    y0 = jnp.floor(ypix)
    fx = xpix - x0
    fy = ypix - y0
    wlv = wl[...]
    hlv = hl[...]
    bsv = base[...]
    for ci, (cx, cy) in enumerate(((0, 0), (1, 0), (0, 1), (1, 1))):
        xi = x0 + cx
        yi = y0 + cy
        valid = ((xi >= 0) & (xi <= wlv - 1) & (yi >= 0) & (yi <= hlv - 1))
        xc = jnp.clip(xi, 0.0, wlv - 1)
        yc = jnp.clip(yi, 0.0, hlv - 1)
        idx = (bsv + yc * wlv + xc) * ROWW
        wxc = fx if cx else 1.0 - fx
        wyc = fy if cy else 1.0 - fy
        w = aw * wxc * wyc * valid.astype(f32)
        wbits = jax.lax.bitcast_convert_type(
            w.astype(jnp.bfloat16), jnp.uint16).astype(jnp.int32)
        comb = (wbits << 16) | idx.astype(jnp.int32)
        comb = jnp.pad(comb, ((0, 0), (0, NPAD - NTOK)))
        for h in range(NH):
            iw_out[0, h, ci] = comb[h * 16:(h + 1) * 16]


def _run_prep(x_all, WqT, WvT, WoxT, WoyT, WaT, bq, bv, box, boy, bat):
    full = lambda s: pl.BlockSpec(s, lambda i: (0,) * len(s))
    grid = (BT,)
    out_shapes = (
        jax.ShapeDtypeStruct((BT, NH, NTOK, ROWW), jnp.float32),
        jax.ShapeDtypeStruct((BT, NH, 4, 16, NPAD), jnp.int32),
    )
    in_specs = [
        pl.BlockSpec((1, D, NTOK), lambda i: (i, 0, 0)),
        full((D, D)), full((D, NH * ROWW)), full((PL, D)), full((PL, D)),
        full((PL, D)),
        full((D, 1)), full((1, NH * ROWW)), full((PL, 1)), full((PL, 1)),
        full((PL, 1)),
        full((PL, NTOK)), full((PL, NTOK)),
        full((PL, 1)), full((PL, 1)), full((PL, 1)), full((PL, PL)),
    ]
    out_specs = (
        pl.BlockSpec((1, NH, NTOK, ROWW), lambda i: (i, 0, 0, 0)),
        pl.BlockSpec((1, NH, 4, 16, NPAD), lambda i: (i, 0, 0, 0, 0)),
    )
    return pl.pallas_call(
        _prep_kernel, grid=grid, in_specs=in_specs, out_specs=out_specs,
        out_shape=out_shapes,
    )(x_all, WqT, WvT, WoxT, WoyT, WaT, bq, bv, box, boy, bat,
      jnp.asarray(_GX.T.copy()), jnp.asarray(_GY.T.copy()),
      jnp.asarray(_WL_LANE.T.copy()), jnp.asarray(_HL_LANE.T.copy()),
      jnp.asarray(_BASE_LANE.T.copy()), jnp.asarray(_SEG))


def _sc_sample_kernel(vh_hbm, iw_hbm, out_hbm, table_v, iw_v0, iw_v1, out_v,
                      sem_t, sem0, sem1):
    nc = 2
    wid = lax.axis_index("s") * nc + lax.axis_index("c")
    bufs = (iw_v0, iw_v1)
    sems = (sem0, sem1)
    ht = pltpu.async_copy(vh_hbm.at[wid], table_v, sem_t)
    handles = [None, None]
    handles[0] = pltpu.async_copy(iw_hbm.at[wid, :, pl.ds(0, QC)], iw_v0, sem0)
    ht.wait()

    for c in range(NCH):
        iw_v = bufs[c % 2]
        handles[c % 2].wait()
        if c + 1 < NCH:
            nxt = (c + 1) % 2
            handles[nxt] = pltpu.async_copy(
                iw_hbm.at[wid, :, pl.ds((c + 1) * QC, QC)], bufs[nxt],
                sems[nxt])

        def group_body(g, carry2, iw_v=iw_v):
            qbase = pl.multiple_of(g * 16, 16)

            def dpass(dlo):
                def j_body(j, accs):
                    iw = iw_v[j, pl.ds(qbase, 16)]
                    idxv = iw & jnp.int32(0xFFFF)
                    wv = plsc.bitcast(iw & jnp.int32(-65536), jnp.float32)
                    out = []
                    for dd in range(16):
                        col = plsc.load_gather(table_v, [idxv + (dlo + dd)])
                        out.append(accs[dd] + wv * col)
                    return tuple(out)

                accs = lax.fori_loop(
                    0, 64, j_body,
                    tuple(jnp.zeros((16,), jnp.float32) for _ in range(16)))
                for dd in range(16):
                    out_v[dlo + dd, pl.ds(qbase, 16)] = accs[dd]

            dpass(0)
            dpass(16)
            return carry2

        lax.fori_loop(0, QG, group_body, 0)
        pltpu.sync_copy(out_v, out_hbm.at[wid, :, pl.ds(c * QC, QC)])


def _run_sc_sample(vh, iw):
    mesh = plsc.VectorSubcoreMesh(core_axis_name="c", subcore_axis_name="s")
    fn = functools.partial(
        pl.kernel,
        out_type=jax.ShapeDtypeStruct((NW, HD, NPAD), jnp.float32),
        mesh=mesh,
        compiler_params=pltpu.CompilerParams(needs_layout_passes=False),
        scratch_types=[
            pltpu.VMEM((NTOK * ROWW,), jnp.float32),
            pltpu.VMEM((64, QC), jnp.int32),
            pltpu.VMEM((64, QC), jnp.int32),
            pltpu.VMEM((HD, QC), jnp.float32),
            pltpu.SemaphoreType.DMA,
            pltpu.SemaphoreType.DMA,
            pltpu.SemaphoreType.DMA,
        ],
    )(_sc_sample_kernel)
    return fn(vh, iw)


def _out_proj_kernel(s_ref, wt, b, o_ref):
    o_ref[0] = jnp.dot(wt[...], s_ref[0],
                       preferred_element_type=jnp.float32) + b[...]


def _run_out_proj(s, Wout, bout_col):
    return pl.pallas_call(
        _out_proj_kernel,
        grid=(BT,),
        in_specs=[
            pl.BlockSpec((1, D, NTOK), lambda i: (i, 0, 0)),
            pl.BlockSpec((D, D), lambda i: (0, 0)),
            pl.BlockSpec((D, 1), lambda i: (0, 0)),
        ],
        out_specs=pl.BlockSpec((1, D, NTOK), lambda i: (i, 0, 0)),
        out_shape=jax.ShapeDtypeStruct((BT, D, NTOK), jnp.float32),
    )(s, Wout, bout_col)


def kernel(feat0, feat1, feat2, feat3, Wq, bq, Wv, bv, Woff, boff,
           Wattn, battn, Wout, bout):
    feats = (feat0, feat1, feat2, feat3)
    x_all = jnp.concatenate(
        [f.transpose(0, 4, 1, 2, 3).reshape(B, T, D, hw)
         for f, hw in zip(feats, HWS)], axis=3).reshape(BT, D, NTOK)

    WvT = jnp.pad(Wv.T.reshape(D, NH, HD), ((0, 0), (0, 0), (0, ROWW - HD)))
    WvT = WvT.reshape(D, NH * ROWW)
    bv_pad = jnp.pad(bv.reshape(NH, HD), ((0, 0), (0, ROWW - HD))).reshape(-1)
    Wox = Woff[0::2]
    Woy = Woff[1::2]

    v, iw = _run_prep(
        x_all, Wq, WvT, Wox, Woy, Wattn,
        bq[:, None], bv_pad[None, :], boff[0::2][:, None],
        boff[1::2][:, None], battn[:, None])

    vh = v.reshape(NW, NTOK * ROWW)

    iw_sc = iw.reshape(NW, 64, NPAD)

    sampled = _run_sc_sample(vh, iw_sc)

    s = sampled.reshape(BT, D, NPAD)[:, :, :NTOK]

    y = _run_out_proj(s, Wout, bout[:, None])
    y = y.reshape(B, T, D, NTOK)

    outs = []
    for l, (h, wdt) in enumerate(LEVEL_HW):
        seg = y[:, :, :, BASES[l]:BASES[l] + HWS[l]]
        seg = seg.reshape(B, T, D, h, wdt).transpose(0, 2, 3, 4, 1)
        outs.append(seg)
    return tuple(outs)
